# Initial kernel scaffold; baseline (speedup 1.0000x reference)
import jax
import jax.numpy as jnp
from jax.experimental import pallas as pl


def kernel(x, tables, W1, b1, W2, b2):
    raise NotImplementedError("write your pallas kernel here")


# speedup vs baseline: 8.1298x; 8.1298x over previous
import functools

import jax
import jax.numpy as jnp
from jax import lax
from jax.experimental import pallas as pl
from jax.experimental.pallas import tpu as pltpu
from jax.experimental.pallas import tpu_sc as plsc

B = 16384
F = 26
V = 100000
D = 32
IN_DIM = F * D
H1 = 256
H2 = 128

_info = plsc.get_sparse_core_info()
NC, NS = _info.num_cores, _info.num_subcores
NW = NC * NS

ROWS_PW = (B * F) // NW
IDX_MINOR = 128
N_IDX_ROWS = ROWS_PW // IDX_MINOR
GPC = 13
CHUNK_ROWS = GPC * IDX_MINOR
N_CHUNKS = N_IDX_ROWS // GPC


@functools.partial(
    pl.kernel,
    mesh=plsc.VectorSubcoreMesh(core_axis_name="c", subcore_axis_name="s"),
    out_type=jax.ShapeDtypeStruct((B * F, D), jnp.float32),
    scratch_types=[
        pltpu.VMEM((N_IDX_ROWS, IDX_MINOR), jnp.int32),
        pltpu.VMEM((CHUNK_ROWS, D), jnp.float32),
        pltpu.SemaphoreType.DMA,
    ],
    compiler_params=pltpu.CompilerParams(use_tc_tiling_on_sc=False),
)
def _sc_gather(idx_hbm, tables_hbm, out_hbm, idx_v, rows_v, sem):
    wid = lax.axis_index("s") * NC + lax.axis_index("c")
    pltpu.sync_copy(idx_hbm.at[wid], idx_v)
    base = wid * ROWS_PW

    def chunk_body(c, carry):
        copies = []
        for j in range(GPC):
            copies.append(pltpu.async_copy(
                tables_hbm.at[idx_v.at[c * GPC + j]],
                rows_v.at[pl.ds(j * IDX_MINOR, IDX_MINOR)],
                sem))
        for cp in copies:
            cp.wait()
        pltpu.sync_copy(rows_v,
                        out_hbm.at[pl.ds(base + c * CHUNK_ROWS, CHUNK_ROWS)])
        return carry

    lax.fori_loop(0, N_CHUNKS, chunk_body, 0)


def _mlp_body(emb_ref, w1_ref, b1_ref, w2_ref, b2_ref, out_ref):
    h = jnp.dot(emb_ref[...], w1_ref[...],
                preferred_element_type=jnp.float32) + b1_ref[...]
    h = jnp.maximum(h, 0.0)
    out_ref[...] = jnp.dot(h, w2_ref[...],
                           preferred_element_type=jnp.float32) + b2_ref[...]


MLP_BLK = 1024


def _mlp(emb, W1, b1, W2, b2):
    return pl.pallas_call(
        _mlp_body,
        grid=(B // MLP_BLK,),
        in_specs=[
            pl.BlockSpec((MLP_BLK, IN_DIM), lambda i: (i, 0)),
            pl.BlockSpec((IN_DIM, H1), lambda i: (0, 0)),
            pl.BlockSpec((1, H1), lambda i: (0, 0)),
            pl.BlockSpec((H1, H2), lambda i: (0, 0)),
            pl.BlockSpec((1, H2), lambda i: (0, 0)),
        ],
        out_specs=pl.BlockSpec((MLP_BLK, H2), lambda i: (i, 0)),
        out_shape=jax.ShapeDtypeStruct((B, H2), jnp.float32),
    )(emb, W1, b1.reshape(1, H1), W2, b2.reshape(1, H2))


def kernel(x, tables, W1, b1, W2, b2):
    xi = x.astype(jnp.int32)
    idx = xi + (jnp.arange(F, dtype=jnp.int32) * V)[None, :]
    idx = idx.reshape(NW, N_IDX_ROWS, IDX_MINOR)
    flat_tables = tables.reshape(F * V, D)
    emb = _sc_gather(idx, flat_tables)
    emb = emb.reshape(B, IN_DIM)
    return _mlp(emb, W1, b1, W2, b2)


# speedup vs baseline: 8.1371x; 1.0009x over previous
import functools

import jax
import jax.numpy as jnp
from jax import lax
from jax.experimental import pallas as pl
from jax.experimental.pallas import tpu as pltpu
from jax.experimental.pallas import tpu_sc as plsc

B = 16384
F = 26
V = 100000
D = 32
IN_DIM = F * D
H1 = 256
H2 = 128

_info = plsc.get_sparse_core_info()
NC, NS = _info.num_cores, _info.num_subcores
NW = NC * NS

B_PW = B // NW
SUB = 128
N_SUB = B_PW // SUB


@functools.partial(
    pl.kernel,
    mesh=plsc.VectorSubcoreMesh(core_axis_name="c", subcore_axis_name="s"),
    out_type=jax.ShapeDtypeStruct((B, IN_DIM), jnp.float32),
    scratch_types=[
        pltpu.VMEM((F * N_SUB, SUB), jnp.int32),
        pltpu.VMEM((2, B_PW, D), jnp.float32),
        pltpu.SemaphoreType.DMA,
        pltpu.SemaphoreType.DMA,
    ],
    compiler_params=pltpu.CompilerParams(use_tc_tiling_on_sc=False),
)
def _sc_gather(idx_hbm, tables_hbm, out_hbm, idx_v, rows_v, sem0, sem1):
    wid = lax.axis_index("s") * NC + lax.axis_index("c")
    pltpu.sync_copy(idx_hbm.at[wid], idx_v)
    base = wid * B_PW
    sems = (sem0, sem1)

    def field_pair(p, carry):
        copies = [[], []]
        for half in range(2):
            f = p * 2 + half
            for s in range(N_SUB):
                copies[half].append(pltpu.async_copy(
                    tables_hbm.at[idx_v.at[f * N_SUB + s]],
                    rows_v.at[half, pl.ds(s * SUB, SUB), :],
                    sems[half]))
        for half in range(2):
            f = p * 2 + half
            for cp in copies[half]:
                cp.wait()
            pltpu.sync_copy(
                rows_v.at[half],
                out_hbm.at[pl.ds(base, B_PW), pl.ds(f * D, D)])
        return carry

    lax.fori_loop(0, F // 2, field_pair, 0)


def _mlp_body(emb_ref, w1_ref, b1_ref, w2_ref, b2_ref, out_ref):
    h = jnp.dot(emb_ref[...], w1_ref[...],
                preferred_element_type=jnp.float32) + b1_ref[...]
    h = jnp.maximum(h, 0.0)
    out_ref[...] = jnp.dot(h, w2_ref[...],
                           preferred_element_type=jnp.float32) + b2_ref[...]


MLP_BLK = 1024


def _mlp(emb, W1, b1, W2, b2):
    return pl.pallas_call(
        _mlp_body,
        grid=(B // MLP_BLK,),
        in_specs=[
            pl.BlockSpec((MLP_BLK, IN_DIM), lambda i: (i, 0)),
            pl.BlockSpec((IN_DIM, H1), lambda i: (0, 0)),
            pl.BlockSpec((1, H1), lambda i: (0, 0)),
            pl.BlockSpec((H1, H2), lambda i: (0, 0)),
            pl.BlockSpec((1, H2), lambda i: (0, 0)),
        ],
        out_specs=pl.BlockSpec((MLP_BLK, H2), lambda i: (i, 0)),
        out_shape=jax.ShapeDtypeStruct((B, H2), jnp.float32),
    )(emb, W1, b1.reshape(1, H1), W2, b2.reshape(1, H2))


def kernel(x, tables, W1, b1, W2, b2):
    xi = x.astype(jnp.int32) + (jnp.arange(F, dtype=jnp.int32) * V)[None, :]
    idx = xi.reshape(NW, N_SUB, SUB, F).transpose(0, 3, 1, 2)
    idx = idx.reshape(NW, F * N_SUB, SUB)
    emb = _sc_gather(idx, tables.reshape(F * V, D))
    return _mlp(emb, W1, b1, W2, b2)


# speedup vs baseline: 8.9991x; 1.1059x over previous
import functools

import jax
import jax.numpy as jnp
from jax import lax
from jax.experimental import pallas as pl
from jax.experimental.pallas import tpu as pltpu
from jax.experimental.pallas import tpu_sc as plsc

B = 16384
F = 26
V = 100000
D = 32
IN_DIM = F * D
H1 = 256
H2 = 128

WIDE = 128
QPR = WIDE // D
VP = 100096
VB = VP // QPR
CBLK = 2176
NCB = VP // CBLK

_info = plsc.get_sparse_core_info()
NC, NS = _info.num_cores, _info.num_subcores
NW = NC * NS

ROWS_PW = (B * F) // NW
IDX_MINOR = 128
N_IDX_ROWS = ROWS_PW // IDX_MINOR
GPC = 13
CHUNK_ROWS = GPC * IDX_MINOR
N_CHUNKS = N_IDX_ROWS // GPC


@functools.partial(
    pl.kernel,
    mesh=plsc.VectorSubcoreMesh(core_axis_name="c", subcore_axis_name="s"),
    out_type=jax.ShapeDtypeStruct((B * F, D), jnp.float32),
    scratch_types=[
        pltpu.VMEM((N_IDX_ROWS, IDX_MINOR), jnp.int32),
        pltpu.VMEM((2, CHUNK_ROWS, D), jnp.float32),
        pltpu.SemaphoreType.DMA,
        pltpu.SemaphoreType.DMA,
    ],
    compiler_params=pltpu.CompilerParams(use_tc_tiling_on_sc=False),
)
def _sc_gather(idx_hbm, tables_hbm, out_hbm, idx_v, rows_v, sem0, sem1):
    wid = lax.axis_index("s") * NC + lax.axis_index("c")
    pltpu.sync_copy(idx_hbm.at[wid], idx_v)
    base = wid * ROWS_PW
    sems = (sem0, sem1)

    def pair_body(i, carry):
        copies = [[], []]
        for h in range(2):
            c = i * 2 + h
            for j in range(GPC):
                copies[h].append(pltpu.async_copy(
                    tables_hbm.at[idx_v.at[c * GPC + j]],
                    rows_v.at[h, pl.ds(j * IDX_MINOR, IDX_MINOR), :],
                    sems[h]))
        for h in range(2):
            c = i * 2 + h
            for cp in copies[h]:
                cp.wait()
            pltpu.sync_copy(
                rows_v.at[h],
                out_hbm.at[pl.ds(base + c * CHUNK_ROWS, CHUNK_ROWS)])
        return carry

    lax.fori_loop(0, N_CHUNKS // 2, pair_body, 0)


SBLK = CBLK // QPR


def _tr_body(t_ref, o_ref):
    blk = t_ref[0]
    t = jnp.swapaxes(blk, 0, 1)
    o_ref[0] = jnp.concatenate(
        [t[q * SBLK:(q + 1) * SBLK] for q in range(QPR)], axis=1)


def _transpose(t3):
    return pl.pallas_call(
        _tr_body,
        grid=(F, NCB),
        in_specs=[pl.BlockSpec((1, D, CBLK), lambda f, c: (f, 0, c))],
        out_specs=pl.BlockSpec((1, CBLK // QPR, WIDE), lambda f, c: (f, c, 0)),
        out_shape=jax.ShapeDtypeStruct((F, VB, WIDE), jnp.float32),
    )(t3)


def _mlp_body(emb_ref, w1_ref, b1_ref, w2_ref, b2_ref, out_ref):
    h = jnp.dot(emb_ref[...], w1_ref[...],
                preferred_element_type=jnp.float32) + b1_ref[...]
    h = jnp.maximum(h, 0.0)
    out_ref[...] = jnp.dot(h, w2_ref[...],
                           preferred_element_type=jnp.float32) + b2_ref[...]


MLP_BLK = 1024


def _mlp(emb, W1, b1, W2, b2):
    return pl.pallas_call(
        _mlp_body,
        grid=(B // MLP_BLK,),
        in_specs=[
            pl.BlockSpec((MLP_BLK, IN_DIM), lambda i: (i, 0)),
            pl.BlockSpec((IN_DIM, H1), lambda i: (0, 0)),
            pl.BlockSpec((1, H1), lambda i: (0, 0)),
            pl.BlockSpec((H1, H2), lambda i: (0, 0)),
            pl.BlockSpec((1, H2), lambda i: (0, 0)),
        ],
        out_specs=pl.BlockSpec((MLP_BLK, H2), lambda i: (i, 0)),
        out_shape=jax.ShapeDtypeStruct((B, H2), jnp.float32),
    )(emb, W1, b1.reshape(1, H1), W2, b2.reshape(1, H2))


def kernel(x, tables, W1, b1, W2, b2):
    xi = x.astype(jnp.int32)
    c = xi // CBLK
    r = xi % CBLK
    s = r // SBLK
    vb = r % SBLK
    fofs = (jnp.arange(F, dtype=jnp.int32) * VB)[None, :]
    idx = (fofs + c * SBLK + vb) * QPR + s
    idx = idx.reshape(NW, N_IDX_ROWS, IDX_MINOR)
    wide = _transpose(jnp.transpose(tables, (0, 2, 1)))
    flat = wide.reshape(F * VB * QPR, D)
    emb = _sc_gather(idx, flat)
    return _mlp(emb.reshape(B, IN_DIM), W1, b1, W2, b2)


# speedup vs baseline: 9.1672x; 1.0187x over previous
import functools

import numpy as np

import jax
import jax.numpy as jnp
from jax import lax
from jax.experimental import pallas as pl
from jax.experimental.pallas import tpu as pltpu
from jax.experimental.pallas import tpu_sc as plsc

B = 16384
F = 26
V = 100000
D = 32
IN_DIM = F * D
H1 = 256
H2 = 128

WIDE = 128
QPR = WIDE // D
VP = 100352
VB = VP // QPR
CBLK = 2048
NCB = VP // CBLK
SBLK = CBLK // QPR

_H = np.zeros((QPR, D, WIDE), np.float32)
for _q in range(QPR):
    _H[_q, :, _q * D:(_q + 1) * D] = np.eye(D, dtype=np.float32)

_info = plsc.get_sparse_core_info()
NC, NS = _info.num_cores, _info.num_subcores
NW = NC * NS

ROWS_PW = (B * F) // NW
IDX_MINOR = 128
N_IDX_ROWS = ROWS_PW // IDX_MINOR
GPC = 13
CHUNK_ROWS = GPC * IDX_MINOR
N_CHUNKS = N_IDX_ROWS // GPC


@functools.partial(
    pl.kernel,
    mesh=plsc.VectorSubcoreMesh(core_axis_name="c", subcore_axis_name="s"),
    out_type=jax.ShapeDtypeStruct((B * F, D), jnp.float32),
    scratch_types=[
        pltpu.VMEM((N_IDX_ROWS, IDX_MINOR), jnp.int32),
        pltpu.VMEM((2, CHUNK_ROWS, D), jnp.float32),
        pltpu.SemaphoreType.DMA,
        pltpu.SemaphoreType.DMA,
    ],
    compiler_params=pltpu.CompilerParams(use_tc_tiling_on_sc=False),
)
def _sc_gather(idx_hbm, tables_hbm, out_hbm, idx_v, rows_v, sem0, sem1):
    wid = lax.axis_index("s") * NC + lax.axis_index("c")
    pltpu.sync_copy(idx_hbm.at[wid], idx_v)
    base = wid * ROWS_PW
    sems = (sem0, sem1)

    def pair_body(i, carry):
        copies = [[], []]
        for h in range(2):
            c = i * 2 + h
            for j in range(GPC):
                copies[h].append(pltpu.async_copy(
                    tables_hbm.at[idx_v.at[c * GPC + j]],
                    rows_v.at[h, pl.ds(j * IDX_MINOR, IDX_MINOR), :],
                    sems[h]))
        for h in range(2):
            c = i * 2 + h
            for cp in copies[h]:
                cp.wait()
            pltpu.sync_copy(
                rows_v.at[h],
                out_hbm.at[pl.ds(base + c * CHUNK_ROWS, CHUNK_ROWS)])
        return carry

    lax.fori_loop(0, N_CHUNKS // 2, pair_body, 0)


def _tr_body(t_ref, h_ref, o_ref):
    blk = t_ref[0]
    acc = None
    for q in range(QPR):
        sub = blk[:, q * SBLK:(q + 1) * SBLK]
        part = jax.lax.dot_general(
            sub, h_ref[q * D:(q + 1) * D, :], (((0,), (0,)), ((), ())),
            preferred_element_type=jnp.float32)
        acc = part if acc is None else acc + part
    o_ref[0] = acc


def _transpose(t3):
    return pl.pallas_call(
        _tr_body,
        grid=(F, NCB),
        in_specs=[
            pl.BlockSpec((1, D, CBLK), lambda f, c: (f, 0, c)),
            pl.BlockSpec((QPR * D, WIDE), lambda f, c: (0, 0)),
        ],
        out_specs=pl.BlockSpec((1, SBLK, WIDE), lambda f, c: (f, c, 0)),
        out_shape=jax.ShapeDtypeStruct((F, VB, WIDE), jnp.float32),
        compiler_params=pltpu.CompilerParams(
            fuse_transposed_lhs_in_matmul=True),
    )(t3, jnp.asarray(_H.reshape(QPR * D, WIDE)))


def _mlp_body(emb_ref, w1_ref, b1_ref, w2_ref, b2_ref, out_ref):
    h = jnp.dot(emb_ref[...], w1_ref[...],
                preferred_element_type=jnp.float32) + b1_ref[...]
    h = jnp.maximum(h, 0.0)
    out_ref[...] = jnp.dot(h, w2_ref[...],
                           preferred_element_type=jnp.float32) + b2_ref[...]


MLP_BLK = 1024


def _mlp(emb, W1, b1, W2, b2):
    return pl.pallas_call(
        _mlp_body,
        grid=(B // MLP_BLK,),
        in_specs=[
            pl.BlockSpec((MLP_BLK, IN_DIM), lambda i: (i, 0)),
            pl.BlockSpec((IN_DIM, H1), lambda i: (0, 0)),
            pl.BlockSpec((1, H1), lambda i: (0, 0)),
            pl.BlockSpec((H1, H2), lambda i: (0, 0)),
            pl.BlockSpec((1, H2), lambda i: (0, 0)),
        ],
        out_specs=pl.BlockSpec((MLP_BLK, H2), lambda i: (i, 0)),
        out_shape=jax.ShapeDtypeStruct((B, H2), jnp.float32),
    )(emb, W1, b1.reshape(1, H1), W2, b2.reshape(1, H2))


def kernel(x, tables, W1, b1, W2, b2):
    xi = x.astype(jnp.int32)
    c = xi >> 11
    s = (xi >> 9) & 3
    vb = xi & 511
    fofs = (jnp.arange(F, dtype=jnp.int32) * VB)[None, :]
    idx = ((fofs + c * SBLK + vb) << 2) + s
    idx = idx.reshape(NW, N_IDX_ROWS, IDX_MINOR)
    wide = _transpose(jnp.transpose(tables, (0, 2, 1)))
    flat = wide.reshape(F * VB * QPR, D)
    emb = _sc_gather(idx, flat)
    return _mlp(emb.reshape(B, IN_DIM), W1, b1, W2, b2)


# speedup vs baseline: 18.8582x; 2.0571x over previous
import functools

import numpy as np

import jax
import jax.numpy as jnp
from jax import lax
from jax.experimental import pallas as pl
from jax.experimental.pallas import tpu as pltpu
from jax.experimental.pallas import tpu_sc as plsc

B = 16384
F = 26
V = 100000
D = 32
IN_DIM = F * D
H1 = 256
H2 = 128

WIDE = 128
QPR = WIDE // D
VP = 100352
VB = VP // QPR
CBLK = 14336
NCB = VP // CBLK
SBLK = CBLK // QPR

_H = np.zeros((QPR, D, WIDE), np.float32)
for _q in range(QPR):
    _H[_q, :, _q * D:(_q + 1) * D] = np.eye(D, dtype=np.float32)

_info = plsc.get_sparse_core_info()
NC, NS = _info.num_cores, _info.num_subcores
NW = NC * NS

ROWS_PW = (B * F) // NW
IDX_MINOR = 128
N_IDX_ROWS = ROWS_PW // IDX_MINOR
GPC = 13
CHUNK_ROWS = GPC * IDX_MINOR
N_CHUNKS = N_IDX_ROWS // GPC


@functools.partial(
    pl.kernel,
    mesh=plsc.VectorSubcoreMesh(core_axis_name="c", subcore_axis_name="s"),
    out_type=jax.ShapeDtypeStruct((B * F, D), jnp.float32),
    scratch_types=[
        pltpu.VMEM((N_IDX_ROWS, IDX_MINOR), jnp.int32),
        pltpu.VMEM((2, CHUNK_ROWS, D), jnp.float32),
        pltpu.SemaphoreType.DMA,
        pltpu.SemaphoreType.DMA,
    ],
    compiler_params=pltpu.CompilerParams(use_tc_tiling_on_sc=False),
)
def _sc_gather(idx_hbm, tables_hbm, out_hbm, idx_v, rows_v, sem0, sem1):
    wid = lax.axis_index("s") * NC + lax.axis_index("c")
    pltpu.sync_copy(idx_hbm.at[wid], idx_v)
    base = wid * ROWS_PW
    sems = (sem0, sem1)

    def pair_body(i, carry):
        copies = [[], []]
        for h in range(2):
            c = i * 2 + h
            for j in range(GPC):
                copies[h].append(pltpu.async_copy(
                    tables_hbm.at[idx_v.at[c * GPC + j]],
                    rows_v.at[h, pl.ds(j * IDX_MINOR, IDX_MINOR), :],
                    sems[h]))
        for h in range(2):
            c = i * 2 + h
            for cp in copies[h]:
                cp.wait()
            pltpu.sync_copy(
                rows_v.at[h],
                out_hbm.at[pl.ds(base + c * CHUNK_ROWS, CHUNK_ROWS)])
        return carry

    lax.fori_loop(0, N_CHUNKS // 2, pair_body, 0)


def _tr_body(t_ref, h_ref, o_ref):
    blk = t_ref[0]
    acc = None
    for q in range(QPR):
        sub = blk[:, q * SBLK:(q + 1) * SBLK]
        part = jax.lax.dot_general(
            sub, h_ref[q * D:(q + 1) * D, :], (((0,), (0,)), ((), ())),
            preferred_element_type=jnp.float32)
        acc = part if acc is None else acc + part
    o_ref[0] = acc


def _transpose(t3):
    return pl.pallas_call(
        _tr_body,
        grid=(F, NCB),
        in_specs=[
            pl.BlockSpec((1, D, CBLK), lambda f, c: (f, 0, c)),
            pl.BlockSpec((QPR * D, WIDE), lambda f, c: (0, 0)),
        ],
        out_specs=pl.BlockSpec((1, SBLK, WIDE), lambda f, c: (f, c, 0)),
        out_shape=jax.ShapeDtypeStruct((F, VB, WIDE), jnp.float32),
        compiler_params=pltpu.CompilerParams(
            fuse_transposed_lhs_in_matmul=True),
    )(t3, jnp.asarray(_H.reshape(QPR * D, WIDE)))


def _mlp_body(emb_ref, w1_ref, b1_ref, w2_ref, b2_ref, out_ref):
    h = jnp.dot(emb_ref[...], w1_ref[...],
                preferred_element_type=jnp.float32) + b1_ref[...]
    h = jnp.maximum(h, 0.0)
    out_ref[...] = jnp.dot(h, w2_ref[...],
                           preferred_element_type=jnp.float32) + b2_ref[...]


MLP_BLK = 1024


def _mlp(emb, W1, b1, W2, b2):
    return pl.pallas_call(
        _mlp_body,
        grid=(B // MLP_BLK,),
        in_specs=[
            pl.BlockSpec((MLP_BLK, IN_DIM), lambda i: (i, 0)),
            pl.BlockSpec((IN_DIM, H1), lambda i: (0, 0)),
            pl.BlockSpec((1, H1), lambda i: (0, 0)),
            pl.BlockSpec((H1, H2), lambda i: (0, 0)),
            pl.BlockSpec((1, H2), lambda i: (0, 0)),
        ],
        out_specs=pl.BlockSpec((MLP_BLK, H2), lambda i: (i, 0)),
        out_shape=jax.ShapeDtypeStruct((B, H2), jnp.float32),
    )(emb, W1, b1.reshape(1, H1), W2, b2.reshape(1, H2))


def kernel(x, tables, W1, b1, W2, b2):
    xi = x.astype(jnp.int32)
    c = xi // CBLK
    r = xi % CBLK
    s = r // SBLK
    vb = r % SBLK
    fofs = (jnp.arange(F, dtype=jnp.int32) * VB)[None, :]
    idx = ((fofs + c * SBLK + vb) << 2) + s
    idx = idx.reshape(NW, N_IDX_ROWS, IDX_MINOR)
    wide = _transpose(jnp.transpose(tables, (0, 2, 1)))
    flat = wide.reshape(F * VB * QPR, D)
    emb = _sc_gather(idx, flat)
    return _mlp(emb.reshape(B, IN_DIM), W1, b1, W2, b2)


# speedup vs baseline: 19.8957x; 1.0550x over previous
import functools

import numpy as np

import jax
import jax.numpy as jnp
from jax import lax
from jax.experimental import pallas as pl
from jax.experimental.pallas import tpu as pltpu
from jax.experimental.pallas import tpu_sc as plsc

B = 16384
F = 26
V = 100000
D = 32
IN_DIM = F * D
H1 = 256
H2 = 128

WIDE = 128
QPR = WIDE // D
VP = 100352
VB = VP // QPR
CBLK = 25088
NCB = VP // CBLK
SBLK = CBLK // QPR

_H = np.zeros((QPR, D, WIDE), np.float32)
for _q in range(QPR):
    _H[_q, :, _q * D:(_q + 1) * D] = np.eye(D, dtype=np.float32)

_info = plsc.get_sparse_core_info()
NC, NS = _info.num_cores, _info.num_subcores
NW = NC * NS

ROWS_PW = (B * F) // NW
IDX_MINOR = 128
N_IDX_ROWS = ROWS_PW // IDX_MINOR
GPC = 13
CHUNK_ROWS = GPC * IDX_MINOR
N_CHUNKS = N_IDX_ROWS // GPC


@functools.partial(
    pl.kernel,
    mesh=plsc.VectorSubcoreMesh(core_axis_name="c", subcore_axis_name="s"),
    out_type=jax.ShapeDtypeStruct((B * F, D), jnp.float32),
    scratch_types=[
        pltpu.VMEM((N_IDX_ROWS, IDX_MINOR), jnp.int32),
        pltpu.VMEM((2, CHUNK_ROWS, D), jnp.float32),
        pltpu.SemaphoreType.DMA,
        pltpu.SemaphoreType.DMA,
    ],
    compiler_params=pltpu.CompilerParams(use_tc_tiling_on_sc=False),
)
def _sc_gather(idx_hbm, tables_hbm, out_hbm, idx_v, rows_v, sem0, sem1):
    wid = lax.axis_index("s") * NC + lax.axis_index("c")
    pltpu.sync_copy(idx_hbm.at[wid], idx_v)
    base = wid * ROWS_PW
    sems = (sem0, sem1)

    def pair_body(i, carry):
        copies = [[], []]
        for h in range(2):
            c = i * 2 + h
            for j in range(GPC):
                copies[h].append(pltpu.async_copy(
                    tables_hbm.at[idx_v.at[c * GPC + j]],
                    rows_v.at[h, pl.ds(j * IDX_MINOR, IDX_MINOR), :],
                    sems[h]))
        for h in range(2):
            c = i * 2 + h
            for cp in copies[h]:
                cp.wait()
            pltpu.sync_copy(
                rows_v.at[h],
                out_hbm.at[pl.ds(base + c * CHUNK_ROWS, CHUNK_ROWS)])
        return carry

    lax.fori_loop(0, N_CHUNKS // 2, pair_body, 0)


def _tr_body(t_ref, h_ref, o_ref):
    blk = t_ref[0]
    acc = None
    for q in range(QPR):
        sub = blk[:, q * SBLK:(q + 1) * SBLK]
        part = jax.lax.dot_general(
            sub, h_ref[q * D:(q + 1) * D, :], (((0,), (0,)), ((), ())),
            preferred_element_type=jnp.float32)
        acc = part if acc is None else acc + part
    o_ref[0] = acc


def _transpose(t3):
    return pl.pallas_call(
        _tr_body,
        grid=(F, NCB),
        in_specs=[
            pl.BlockSpec((1, D, CBLK), lambda f, c: (f, 0, c)),
            pl.BlockSpec((QPR * D, WIDE), lambda f, c: (0, 0)),
        ],
        out_specs=pl.BlockSpec((1, SBLK, WIDE), lambda f, c: (f, c, 0)),
        out_shape=jax.ShapeDtypeStruct((F, VB, WIDE), jnp.float32),
        compiler_params=pltpu.CompilerParams(
            fuse_transposed_lhs_in_matmul=True),
    )(t3, jnp.asarray(_H.reshape(QPR * D, WIDE)))


def _mlp_body(emb_ref, w1_ref, b1_ref, w2_ref, b2_ref, out_ref):
    h = jnp.dot(emb_ref[...], w1_ref[...],
                preferred_element_type=jnp.float32) + b1_ref[...]
    h = jnp.maximum(h, 0.0)
    out_ref[...] = jnp.dot(h, w2_ref[...],
                           preferred_element_type=jnp.float32) + b2_ref[...]


MLP_BLK = 1024


def _mlp(emb, W1, b1, W2, b2):
    return pl.pallas_call(
        _mlp_body,
        grid=(B // MLP_BLK,),
        in_specs=[
            pl.BlockSpec((MLP_BLK, IN_DIM), lambda i: (i, 0)),
            pl.BlockSpec((IN_DIM, H1), lambda i: (0, 0)),
            pl.BlockSpec((1, H1), lambda i: (0, 0)),
            pl.BlockSpec((H1, H2), lambda i: (0, 0)),
            pl.BlockSpec((1, H2), lambda i: (0, 0)),
        ],
        out_specs=pl.BlockSpec((MLP_BLK, H2), lambda i: (i, 0)),
        out_shape=jax.ShapeDtypeStruct((B, H2), jnp.float32),
    )(emb, W1, b1.reshape(1, H1), W2, b2.reshape(1, H2))


def kernel(x, tables, W1, b1, W2, b2):
    xi = x.astype(jnp.int32)
    c = xi // CBLK
    r = xi % CBLK
    s = r // SBLK
    vb = r % SBLK
    fofs = (jnp.arange(F, dtype=jnp.int32) * VB)[None, :]
    idx = ((fofs + c * SBLK + vb) << 2) + s
    idx = idx.reshape(NW, N_IDX_ROWS, IDX_MINOR)
    wide = _transpose(jnp.transpose(tables, (0, 2, 1)))
    flat = wide.reshape(F * VB * QPR, D)
    emb = _sc_gather(idx, flat)
    return _mlp(emb.reshape(B, IN_DIM), W1, b1, W2, b2)


# speedup vs baseline: 20.2342x; 1.0170x over previous
import functools

import numpy as np

import jax
import jax.numpy as jnp
from jax import lax
from jax.experimental import pallas as pl
from jax.experimental.pallas import tpu as pltpu
from jax.experimental.pallas import tpu_sc as plsc

B = 16384
F = 26
V = 100000
D = 32
IN_DIM = F * D
H1 = 256
H2 = 128

WIDE = 128
QPR = WIDE // D
VP = 100352
VB = VP // QPR
CBLK = 50176
NCB = VP // CBLK
SBLK = CBLK // QPR

_H = np.zeros((QPR, D, WIDE), np.float32)
for _q in range(QPR):
    _H[_q, :, _q * D:(_q + 1) * D] = np.eye(D, dtype=np.float32)

_info = plsc.get_sparse_core_info()
NC, NS = _info.num_cores, _info.num_subcores
NW = NC * NS

ROWS_PW = (B * F) // NW
IDX_MINOR = 128
N_IDX_ROWS = ROWS_PW // IDX_MINOR
GPC = 13
CHUNK_ROWS = GPC * IDX_MINOR
N_CHUNKS = N_IDX_ROWS // GPC


@functools.partial(
    pl.kernel,
    mesh=plsc.VectorSubcoreMesh(core_axis_name="c", subcore_axis_name="s"),
    out_type=jax.ShapeDtypeStruct((B * F, D), jnp.float32),
    scratch_types=[
        pltpu.VMEM((N_IDX_ROWS, IDX_MINOR), jnp.int32),
        pltpu.VMEM((2, CHUNK_ROWS, D), jnp.float32),
        pltpu.SemaphoreType.DMA,
        pltpu.SemaphoreType.DMA,
    ],
    compiler_params=pltpu.CompilerParams(use_tc_tiling_on_sc=False),
)
def _sc_gather(idx_hbm, tables_hbm, out_hbm, idx_v, rows_v, sem0, sem1):
    wid = lax.axis_index("s") * NC + lax.axis_index("c")
    pltpu.sync_copy(idx_hbm.at[wid], idx_v)
    base = wid * ROWS_PW
    sems = (sem0, sem1)

    def pair_body(i, carry):
        copies = [[], []]
        for h in range(2):
            c = i * 2 + h
            for j in range(GPC):
                copies[h].append(pltpu.async_copy(
                    tables_hbm.at[idx_v.at[c * GPC + j]],
                    rows_v.at[h, pl.ds(j * IDX_MINOR, IDX_MINOR), :],
                    sems[h]))
        for h in range(2):
            c = i * 2 + h
            for cp in copies[h]:
                cp.wait()
            pltpu.sync_copy(
                rows_v.at[h],
                out_hbm.at[pl.ds(base + c * CHUNK_ROWS, CHUNK_ROWS)])
        return carry

    lax.fori_loop(0, N_CHUNKS // 2, pair_body, 0)


def _tr_body(t_ref, h_ref, o_ref):
    blk = t_ref[0]
    acc = None
    for q in range(QPR):
        sub = blk[:, q * SBLK:(q + 1) * SBLK]
        part = jax.lax.dot_general(
            sub, h_ref[q * D:(q + 1) * D, :], (((0,), (0,)), ((), ())),
            preferred_element_type=jnp.float32)
        acc = part if acc is None else acc + part
    o_ref[0] = acc


def _transpose(t3):
    return pl.pallas_call(
        _tr_body,
        grid=(F, NCB),
        in_specs=[
            pl.BlockSpec((1, D, CBLK), lambda f, c: (f, 0, c)),
            pl.BlockSpec((QPR * D, WIDE), lambda f, c: (0, 0)),
        ],
        out_specs=pl.BlockSpec((1, SBLK, WIDE), lambda f, c: (f, c, 0)),
        out_shape=jax.ShapeDtypeStruct((F, VB, WIDE), jnp.float32),
        compiler_params=pltpu.CompilerParams(
            fuse_transposed_lhs_in_matmul=True),
    )(t3, jnp.asarray(_H.reshape(QPR * D, WIDE)))


def _mlp_body(emb_ref, w1_ref, b1_ref, w2_ref, b2_ref, out_ref):
    h = jnp.dot(emb_ref[...], w1_ref[...],
                preferred_element_type=jnp.float32) + b1_ref[...]
    h = jnp.maximum(h, 0.0)
    out_ref[...] = jnp.dot(h, w2_ref[...],
                           preferred_element_type=jnp.float32) + b2_ref[...]


MLP_BLK = 1024


def _mlp(emb, W1, b1, W2, b2):
    return pl.pallas_call(
        _mlp_body,
        grid=(B // MLP_BLK,),
        in_specs=[
            pl.BlockSpec((MLP_BLK, IN_DIM), lambda i: (i, 0)),
            pl.BlockSpec((IN_DIM, H1), lambda i: (0, 0)),
            pl.BlockSpec((1, H1), lambda i: (0, 0)),
            pl.BlockSpec((H1, H2), lambda i: (0, 0)),
            pl.BlockSpec((1, H2), lambda i: (0, 0)),
        ],
        out_specs=pl.BlockSpec((MLP_BLK, H2), lambda i: (i, 0)),
        out_shape=jax.ShapeDtypeStruct((B, H2), jnp.float32),
    )(emb, W1, b1.reshape(1, H1), W2, b2.reshape(1, H2))


def kernel(x, tables, W1, b1, W2, b2):
    xi = x.astype(jnp.int32)
    c = xi // CBLK
    r = xi % CBLK
    s = r // SBLK
    vb = r % SBLK
    fofs = (jnp.arange(F, dtype=jnp.int32) * VB)[None, :]
    idx = ((fofs + c * SBLK + vb) << 2) + s
    idx = idx.reshape(NW, N_IDX_ROWS, IDX_MINOR)
    wide = _transpose(jnp.transpose(tables, (0, 2, 1)))
    flat = wide.reshape(F * VB * QPR, D)
    emb = _sc_gather(idx, flat)
    return _mlp(emb.reshape(B, IN_DIM), W1, b1, W2, b2)
